# baseline (speedup 1.0000x reference)
import functools

import jax
import jax.numpy as jnp
from jax import lax
from jax.experimental import pallas as pl
from jax.experimental.pallas import tpu as pltpu
from jax.experimental.pallas import tpu_sc as plsc

BETA = 0.2
K = 8192
NQ = 12544
NT = 128
NW = 32
NPAD = 12800
BPW = NPAD // NW
CHUNK = 16


def _vq_argmin_body(qf_ref, cbt_ref, idx_ref):
    q = qf_ref[...]
    cbt = cbt_ref[...]
    f32 = jnp.float32
    cbx = cbt[0:1, :]
    cby = cbt[1:2, :]
    qx = q[:, 0:1]
    qy = q[:, 1:2]
    b2 = cbx * cbx + cby * cby
    a2 = qx * qx + qy * qy
    m = (qx.astype(jnp.bfloat16).astype(f32) * cbx.astype(jnp.bfloat16).astype(f32)
         + qy.astype(jnp.bfloat16).astype(f32) * cby.astype(jnp.bfloat16).astype(f32))
    dist = jnp.sqrt(jnp.maximum((a2 + b2) - 2.0 * m, 0.0))
    mn = jnp.min(dist, axis=1, keepdims=True)
    ii = lax.broadcasted_iota(jnp.int32, dist.shape, 1)
    idx = jnp.min(jnp.where(dist <= mn, ii, K), axis=1)
    idx_ref[...] = idx.astype(jnp.int32).reshape(NT, 1)


def _diag_argmin_body(s_ref, idx_ref):
    score = s_ref[...]
    mn = jnp.min(score, axis=1, keepdims=True)
    ii = lax.broadcasted_iota(jnp.int32, score.shape, 1)
    idx = jnp.min(jnp.where(score <= mn, ii, K), axis=1)
    idx_ref[...] = idx.astype(jnp.int32).reshape(NT, 1)


def _diag_argmin(dist):
    return pl.pallas_call(
        _diag_argmin_body,
        grid=(NQ // NT,),
        in_specs=[pl.BlockSpec((NT, K), lambda i: (i, 0))],
        out_specs=pl.BlockSpec((NT, 1), lambda i: (i, 0)),
        out_shape=jax.ShapeDtypeStruct((NQ, 1), jnp.int32),
    )(dist)


def _vq_argmin(qf, cbt):
    return pl.pallas_call(
        _vq_argmin_body,
        grid=(NQ // NT,),
        in_specs=[
            pl.BlockSpec((NT, 2), lambda i: (i, 0)),
            pl.BlockSpec((2, K), lambda i: (0, 0)),
        ],
        out_specs=pl.BlockSpec((NT, 1), lambda i: (i, 0)),
        out_shape=jax.ShapeDtypeStruct((NQ, 1), jnp.int32),
    )(qf, cbt)


def _sc_quantize(cbx, cby, idx, qfx, qfy):
    mesh = plsc.VectorSubcoreMesh(core_axis_name="c", subcore_axis_name="s")

    @functools.partial(
        pl.kernel,
        mesh=mesh,
        compiler_params=pltpu.CompilerParams(needs_layout_passes=False),
        out_type=[
            jax.ShapeDtypeStruct((NPAD,), jnp.float32),
            jax.ShapeDtypeStruct((NPAD,), jnp.float32),
        ],
        scratch_types=[
            pltpu.VMEM((K,), jnp.float32),
            pltpu.VMEM((K,), jnp.float32),
            pltpu.VMEM((BPW,), jnp.int32),
            pltpu.VMEM((BPW,), jnp.float32),
            pltpu.VMEM((BPW,), jnp.float32),
            pltpu.VMEM((BPW,), jnp.float32),
            pltpu.VMEM((BPW,), jnp.float32),
        ],
    )
    def gather_kernel(cbx_hbm, cby_hbm, idx_hbm, qfx_hbm, qfy_hbm,
                      ox_hbm, oy_hbm,
                      cbx_v, cby_v, idx_v, qfx_v, qfy_v, ox_v, oy_v):
        wid = lax.axis_index("s") * 2 + lax.axis_index("c")
        base = wid * BPW
        pltpu.sync_copy(cbx_hbm, cbx_v)
        pltpu.sync_copy(cby_hbm, cby_v)
        pltpu.sync_copy(idx_hbm.at[pl.ds(base, BPW)], idx_v)
        pltpu.sync_copy(qfx_hbm.at[pl.ds(base, BPW)], qfx_v)
        pltpu.sync_copy(qfy_hbm.at[pl.ds(base, BPW)], qfy_v)

        def body(i, carry):
            sl = pl.ds(i * CHUNK, CHUNK)
            iv = idx_v[sl]
            gx = plsc.load_gather(cbx_v, [iv])
            gy = plsc.load_gather(cby_v, [iv])
            fx = qfx_v[sl]
            fy = qfy_v[sl]
            ox_v[sl] = fx + (gx - fx)
            oy_v[sl] = fy + (gy - fy)
            return carry

        lax.fori_loop(0, BPW // CHUNK, body, 0)
        pltpu.sync_copy(ox_v, ox_hbm.at[pl.ds(base, BPW)])
        pltpu.sync_copy(oy_v, oy_hbm.at[pl.ds(base, BPW)])

    return gather_kernel(cbx, cby, idx, qfx, qfy)


def _loss_body(out_ref, x_ref, qst_ref, qf_ref, recon_ref, cb_ref, tot_ref):
    d = out_ref[...] - x_ref[...]
    recon = jnp.sum(d * d) / jnp.float32(out_ref.shape[0] * out_ref.shape[1])
    e = qst_ref[...] - qf_ref[...]
    cb = jnp.sum(e * e) / jnp.float32(qst_ref.shape[0] * qst_ref.shape[1])
    recon_ref[0, 0] = recon
    cb_ref[0, 0] = cb
    tot_ref[0, 0] = recon + (cb + BETA * cb)


def _losses(out2, x2, qst2, qf2):
    s = jax.ShapeDtypeStruct((1, 1), jnp.float32)
    smem = pl.BlockSpec(memory_space=pltpu.SMEM)
    return pl.pallas_call(
        _loss_body,
        out_specs=[smem, smem, smem],
        out_shape=[s, s, s],
    )(out2, x2, qst2, qf2)


def _conv(x, w, b, stride, pad):
    y = lax.conv_general_dilated(x, w, (stride, stride), [(pad, pad), (pad, pad)],
                                 dimension_numbers=('NCHW', 'OIHW', 'NCHW'))
    return y + b[None, :, None, None]


def _conv_t(x, w, b, stride, pad):
    k = w.shape[2]
    w2 = jnp.flip(w, axis=(2, 3)).transpose(1, 0, 2, 3)
    p = k - 1 - pad
    y = lax.conv_general_dilated(x, w2, (1, 1), [(p, p), (p, p)],
                                 lhs_dilation=(stride, stride),
                                 dimension_numbers=('NCHW', 'OIHW', 'NCHW'))
    return y + b[None, :, None, None]


def _bn(x, g, b):
    m = jnp.mean(x, axis=(0, 2, 3), keepdims=True)
    v = jnp.var(x, axis=(0, 2, 3), keepdims=True)
    return (x - m) / jnp.sqrt(v + 1e-5) * g[None, :, None, None] + b[None, :, None, None]


def kernel(x, enc_w1, enc_b1, bn1_g, bn1_b, enc_w2, enc_b2, bn2_g, bn2_b,
           preq_w, preq_b, codebook, postq_w, postq_b, dec_w1, dec_b1,
           bn3_g, bn3_b, dec_w2, dec_b2):
    h = jax.nn.relu(_bn(_conv(x, enc_w1, enc_b1, 2, 1), bn1_g, bn1_b))
    h = jax.nn.relu(_bn(_conv(h, enc_w2, enc_b2, 2, 1), bn2_g, bn2_b))
    q_in = _conv(h, preq_w, preq_b, 1, 0)
    Bq, C, Hq, Wq = q_in.shape
    qf = q_in.transpose(0, 2, 3, 1).reshape(NQ, C)

    cbt = codebook.T
    idx = _vq_argmin(qf, cbt).reshape(NQ)

    pad = NPAD - NQ
    idx_p = jnp.concatenate([idx, jnp.zeros((pad,), jnp.int32)])
    qfx_p = jnp.concatenate([qf[:, 0], jnp.zeros((pad,), jnp.float32)])
    qfy_p = jnp.concatenate([qf[:, 1], jnp.zeros((pad,), jnp.float32)])
    qstx, qsty = _sc_quantize(cbt[0], cbt[1], idx_p, qfx_p, qfy_p)
    qst = jnp.stack([qstx[:NQ], qsty[:NQ]], axis=-1)
    q_out = qst.reshape(Bq, Hq, Wq, C).transpose(0, 3, 1, 2)

    d = _conv(q_out, postq_w, postq_b, 1, 0)
    d = jax.nn.relu(_bn(_conv_t(d, dec_w1, dec_b1, 2, 1), bn3_g, bn3_b))
    out = jnp.tanh(_conv_t(d, dec_w2, dec_b2, 2, 1))

    recon2, cb2, tot2 = _losses(out.reshape(1568, 128), x.reshape(1568, 128),
                                qst.reshape(196, 128), qf.reshape(196, 128))
    recon = recon2[0, 0]
    cb_loss = cb2[0, 0]
    total = tot2[0, 0]
    return (out, total, (recon, cb_loss, cb_loss))


# speedup vs baseline: 1.0954x; 1.0954x over previous
import functools

import jax
import jax.numpy as jnp
from jax import lax
from jax.experimental import pallas as pl
from jax.experimental.pallas import tpu as pltpu
from jax.experimental.pallas import tpu_sc as plsc

BETA = 0.2
K = 8192
NQ = 12544
NT = 256
NW = 32
NPAD = 12800
BPW = NPAD // NW
CHUNK = 16
NPIX = 200704
PPW = NPIX // NW


def _vq_argmin_body(qf_ref, cbt_ref, idx_ref):
    f32 = jnp.float32
    q = qf_ref[...]
    cbt = cbt_ref[...]
    cbx = cbt[0:1, :]
    cby = cbt[1:2, :]
    qx = q[:, 0:1]
    qy = q[:, 1:2]
    b2 = cbx * cbx + cby * cby
    a2 = qx * qx + qy * qy
    m = lax.dot_general(q.astype(jnp.bfloat16), cbt.astype(jnp.bfloat16),
                        (((1,), (0,)), ((), ())),
                        preferred_element_type=f32)
    dist = jnp.sqrt(jnp.maximum((a2 + b2) - 2.0 * m, 0.0))
    mn = jnp.min(dist, axis=1, keepdims=True)
    ii = lax.broadcasted_iota(jnp.int32, dist.shape, 1)
    idx = jnp.min(jnp.where(dist <= mn, ii, K), axis=1)
    idx_ref[...] = idx.astype(jnp.int32).reshape(NT, 1)


def _vq_argmin(qf, cbt):
    return pl.pallas_call(
        _vq_argmin_body,
        grid=(NQ // NT,),
        in_specs=[
            pl.BlockSpec((NT, 2), lambda i: (i, 0)),
            pl.BlockSpec((2, K), lambda i: (0, 0)),
        ],
        out_specs=pl.BlockSpec((NT, 1), lambda i: (i, 0)),
        out_shape=jax.ShapeDtypeStruct((NQ, 1), jnp.int32),
    )(qf, cbt)


def _sc_quantize(cbx, cby, idx, qfx, qfy):
    mesh = plsc.VectorSubcoreMesh(core_axis_name="c", subcore_axis_name="s")

    @functools.partial(
        pl.kernel,
        mesh=mesh,
        compiler_params=pltpu.CompilerParams(needs_layout_passes=False),
        out_type=[
            jax.ShapeDtypeStruct((NPAD,), jnp.float32),
            jax.ShapeDtypeStruct((NPAD,), jnp.float32),
            jax.ShapeDtypeStruct((NW, CHUNK), jnp.float32),
        ],
        scratch_types=[
            pltpu.VMEM((K,), jnp.float32),
            pltpu.VMEM((K,), jnp.float32),
            pltpu.VMEM((BPW,), jnp.int32),
            pltpu.VMEM((BPW,), jnp.float32),
            pltpu.VMEM((BPW,), jnp.float32),
            pltpu.VMEM((BPW,), jnp.float32),
            pltpu.VMEM((BPW,), jnp.float32),
            pltpu.VMEM((CHUNK,), jnp.float32),
        ],
    )
    def gather_kernel(cbx_hbm, cby_hbm, idx_hbm, qfx_hbm, qfy_hbm,
                      ox_hbm, oy_hbm, part_hbm,
                      cbx_v, cby_v, idx_v, qfx_v, qfy_v, ox_v, oy_v, acc_v):
        wid = lax.axis_index("s") * 2 + lax.axis_index("c")
        base = wid * BPW
        pltpu.sync_copy(cbx_hbm, cbx_v)
        pltpu.sync_copy(cby_hbm, cby_v)
        pltpu.sync_copy(idx_hbm.at[pl.ds(base, BPW)], idx_v)
        pltpu.sync_copy(qfx_hbm.at[pl.ds(base, BPW)], qfx_v)
        pltpu.sync_copy(qfy_hbm.at[pl.ds(base, BPW)], qfy_v)
        acc_v[...] = jnp.zeros((CHUNK,), jnp.float32)

        def body(i, carry):
            sl = pl.ds(i * CHUNK, CHUNK)
            iv = idx_v[sl]
            gx = plsc.load_gather(cbx_v, [iv])
            gy = plsc.load_gather(cby_v, [iv])
            fx = qfx_v[sl]
            fy = qfy_v[sl]
            valid = (base + i * CHUNK + lax.iota(jnp.int32, CHUNK)) < NQ
            zero = jnp.zeros((CHUNK,), jnp.float32)
            ex = jnp.where(valid, gx - fx, zero)
            ey = jnp.where(valid, gy - fy, zero)
            acc_v[...] = acc_v[...] + (ex * ex + ey * ey)
            ox_v[sl] = jnp.where(valid, fx + ex, zero)
            oy_v[sl] = jnp.where(valid, fy + ey, zero)
            return carry

        lax.fori_loop(0, BPW // CHUNK, body, 0)
        pltpu.sync_copy(ox_v, ox_hbm.at[pl.ds(base, BPW)])
        pltpu.sync_copy(oy_v, oy_hbm.at[pl.ds(base, BPW)])
        pltpu.sync_copy(acc_v, part_hbm.at[wid])

    return gather_kernel(cbx, cby, idx, qfx, qfy)


def _sc_sqdiff(a, b):
    mesh = plsc.VectorSubcoreMesh(core_axis_name="c", subcore_axis_name="s")

    @functools.partial(
        pl.kernel,
        mesh=mesh,
        compiler_params=pltpu.CompilerParams(needs_layout_passes=False),
        out_type=jax.ShapeDtypeStruct((NW, CHUNK), jnp.float32),
        scratch_types=[
            pltpu.VMEM((PPW,), jnp.float32),
            pltpu.VMEM((PPW,), jnp.float32),
            pltpu.VMEM((CHUNK,), jnp.float32),
        ],
    )
    def sqdiff_kernel(a_hbm, b_hbm, part_hbm, a_v, b_v, acc_v):
        wid = lax.axis_index("s") * 2 + lax.axis_index("c")
        base = wid * PPW
        pltpu.sync_copy(a_hbm.at[pl.ds(base, PPW)], a_v)
        pltpu.sync_copy(b_hbm.at[pl.ds(base, PPW)], b_v)
        acc_v[...] = jnp.zeros((CHUNK,), jnp.float32)

        def body(i, carry):
            sl = pl.ds(i * CHUNK, CHUNK)
            d = a_v[sl] - b_v[sl]
            acc_v[...] = acc_v[...] + d * d
            return carry

        lax.fori_loop(0, PPW // CHUNK, body, 0)
        pltpu.sync_copy(acc_v, part_hbm.at[wid])

    return sqdiff_kernel(a, b)


def _conv(x, w, b, stride, pad):
    y = lax.conv_general_dilated(x, w, (stride, stride), [(pad, pad), (pad, pad)],
                                 dimension_numbers=('NCHW', 'OIHW', 'NCHW'))
    return y + b[None, :, None, None]


def _conv_t(x, w, b, stride, pad):
    k = w.shape[2]
    w2 = jnp.flip(w, axis=(2, 3)).transpose(1, 0, 2, 3)
    p = k - 1 - pad
    y = lax.conv_general_dilated(x, w2, (1, 1), [(p, p), (p, p)],
                                 lhs_dilation=(stride, stride),
                                 dimension_numbers=('NCHW', 'OIHW', 'NCHW'))
    return y + b[None, :, None, None]


def _bn(x, g, b):
    m = jnp.mean(x, axis=(0, 2, 3), keepdims=True)
    v = jnp.var(x, axis=(0, 2, 3), keepdims=True)
    return (x - m) / jnp.sqrt(v + 1e-5) * g[None, :, None, None] + b[None, :, None, None]


def kernel(x, enc_w1, enc_b1, bn1_g, bn1_b, enc_w2, enc_b2, bn2_g, bn2_b,
           preq_w, preq_b, codebook, postq_w, postq_b, dec_w1, dec_b1,
           bn3_g, bn3_b, dec_w2, dec_b2):
    h = jax.nn.relu(_bn(_conv(x, enc_w1, enc_b1, 2, 1), bn1_g, bn1_b))
    h = jax.nn.relu(_bn(_conv(h, enc_w2, enc_b2, 2, 1), bn2_g, bn2_b))
    q_in = _conv(h, preq_w, preq_b, 1, 0)
    Bq, C, Hq, Wq = q_in.shape
    qf = q_in.transpose(0, 2, 3, 1).reshape(NQ, C)

    cbt = codebook.T
    idx = _vq_argmin(qf, cbt).reshape(NQ)

    pad = NPAD - NQ
    idx_p = jnp.concatenate([idx, jnp.zeros((pad,), jnp.int32)])
    qfx_p = jnp.concatenate([qf[:, 0], jnp.zeros((pad,), jnp.float32)])
    qfy_p = jnp.concatenate([qf[:, 1], jnp.zeros((pad,), jnp.float32)])
    qstx, qsty, cb_part = _sc_quantize(cbt[0], cbt[1], idx_p, qfx_p, qfy_p)
    qst = jnp.stack([qstx[:NQ], qsty[:NQ]], axis=-1)
    q_out = qst.reshape(Bq, Hq, Wq, C).transpose(0, 3, 1, 2)

    d = _conv(q_out, postq_w, postq_b, 1, 0)
    d = jax.nn.relu(_bn(_conv_t(d, dec_w1, dec_b1, 2, 1), bn3_g, bn3_b))
    out = jnp.tanh(_conv_t(d, dec_w2, dec_b2, 2, 1))

    r_part = _sc_sqdiff(out.reshape(NPIX), x.reshape(NPIX))
    recon = jnp.sum(r_part) / jnp.float32(NPIX)
    cb_loss = jnp.sum(cb_part) / jnp.float32(2 * NQ)
    total = recon + (cb_loss + BETA * cb_loss)
    return (out, total, (recon, cb_loss, cb_loss))


# speedup vs baseline: 1.5492x; 1.4143x over previous
import functools

import jax
import jax.numpy as jnp
from jax import lax
from jax.experimental import pallas as pl
from jax.experimental.pallas import tpu as pltpu
from jax.experimental.pallas import tpu_sc as plsc

BETA = 0.2
K = 8192
NQ = 12544
NT = 256
NW = 32
NPAD = 12800
BPW = NPAD // NW
CHUNK = 16
NPIX = 200704
PPW = NPIX // NW


def _vq_argmin_body(qf_ref, cbt_ref, idx_ref):
    f32 = jnp.float32
    q = qf_ref[...]
    cbt = cbt_ref[...]
    cbx = cbt[0:1, :]
    cby = cbt[1:2, :]
    qx = q[:, 0:1]
    qy = q[:, 1:2]
    b2 = cbx * cbx + cby * cby
    a2 = qx * qx + qy * qy
    m = lax.dot_general(q.astype(jnp.bfloat16), cbt.astype(jnp.bfloat16),
                        (((1,), (0,)), ((), ())),
                        preferred_element_type=f32)
    dist = jnp.sqrt(jnp.maximum((a2 + b2) - 2.0 * m, 0.0))
    mn = jnp.min(dist, axis=1, keepdims=True)
    ii = lax.broadcasted_iota(jnp.int32, dist.shape, 1)
    idx = jnp.min(jnp.where(dist <= mn, ii, K), axis=1)
    idx_ref[...] = idx.astype(jnp.int32).reshape(NT, 1)


def _vq_argmin(qf, cbt):
    return pl.pallas_call(
        _vq_argmin_body,
        grid=(NQ // NT,),
        in_specs=[
            pl.BlockSpec((NT, 2), lambda i: (i, 0)),
            pl.BlockSpec((2, K), lambda i: (0, 0)),
        ],
        out_specs=pl.BlockSpec((NT, 1), lambda i: (i, 0)),
        out_shape=jax.ShapeDtypeStruct((NQ, 1), jnp.int32),
    )(qf, cbt)


def _sc_quantize(cbx, cby, idx, qfx, qfy):
    mesh = plsc.VectorSubcoreMesh(core_axis_name="c", subcore_axis_name="s")

    @functools.partial(
        pl.kernel,
        mesh=mesh,
        compiler_params=pltpu.CompilerParams(needs_layout_passes=False),
        out_type=[
            jax.ShapeDtypeStruct((NPAD,), jnp.float32),
            jax.ShapeDtypeStruct((NPAD,), jnp.float32),
            jax.ShapeDtypeStruct((NW, CHUNK), jnp.float32),
        ],
        scratch_types=[
            pltpu.VMEM((K,), jnp.float32),
            pltpu.VMEM((K,), jnp.float32),
            pltpu.VMEM((BPW,), jnp.int32),
            pltpu.VMEM((BPW,), jnp.float32),
            pltpu.VMEM((BPW,), jnp.float32),
            pltpu.VMEM((BPW,), jnp.float32),
            pltpu.VMEM((BPW,), jnp.float32),
            pltpu.VMEM((CHUNK,), jnp.float32),
        ],
    )
    def gather_kernel(cbx_hbm, cby_hbm, idx_hbm, qfx_hbm, qfy_hbm,
                      ox_hbm, oy_hbm, part_hbm,
                      cbx_v, cby_v, idx_v, qfx_v, qfy_v, ox_v, oy_v, acc_v):
        wid = lax.axis_index("s") * 2 + lax.axis_index("c")
        base = wid * BPW
        pltpu.sync_copy(cbx_hbm, cbx_v)
        pltpu.sync_copy(cby_hbm, cby_v)
        pltpu.sync_copy(idx_hbm.at[pl.ds(base, BPW)], idx_v)
        pltpu.sync_copy(qfx_hbm.at[pl.ds(base, BPW)], qfx_v)
        pltpu.sync_copy(qfy_hbm.at[pl.ds(base, BPW)], qfy_v)
        acc_v[...] = jnp.zeros((CHUNK,), jnp.float32)

        def body(i, carry):
            sl = pl.ds(i * CHUNK, CHUNK)
            iv = idx_v[sl]
            gx = plsc.load_gather(cbx_v, [iv])
            gy = plsc.load_gather(cby_v, [iv])
            fx = qfx_v[sl]
            fy = qfy_v[sl]
            valid = (base + i * CHUNK + lax.iota(jnp.int32, CHUNK)) < NQ
            zero = jnp.zeros((CHUNK,), jnp.float32)
            ex = jnp.where(valid, gx - fx, zero)
            ey = jnp.where(valid, gy - fy, zero)
            acc_v[...] = acc_v[...] + (ex * ex + ey * ey)
            ox_v[sl] = jnp.where(valid, fx + ex, zero)
            oy_v[sl] = jnp.where(valid, fy + ey, zero)
            return carry

        lax.fori_loop(0, BPW // CHUNK, body, 0)
        pltpu.sync_copy(ox_v, ox_hbm.at[pl.ds(base, BPW)])
        pltpu.sync_copy(oy_v, oy_hbm.at[pl.ds(base, BPW)])
        pltpu.sync_copy(acc_v, part_hbm.at[wid])

    return gather_kernel(cbx, cby, idx, qfx, qfy)


def _sc_sqdiff(a, b):
    mesh = plsc.VectorSubcoreMesh(core_axis_name="c", subcore_axis_name="s")

    @functools.partial(
        pl.kernel,
        mesh=mesh,
        compiler_params=pltpu.CompilerParams(needs_layout_passes=False),
        out_type=jax.ShapeDtypeStruct((NW, CHUNK), jnp.float32),
        scratch_types=[
            pltpu.VMEM((PPW,), jnp.float32),
            pltpu.VMEM((PPW,), jnp.float32),
            pltpu.VMEM((CHUNK,), jnp.float32),
        ],
    )
    def sqdiff_kernel(a_hbm, b_hbm, part_hbm, a_v, b_v, acc_v):
        wid = lax.axis_index("s") * 2 + lax.axis_index("c")
        base = wid * PPW
        pltpu.sync_copy(a_hbm.at[pl.ds(base, PPW)], a_v)
        pltpu.sync_copy(b_hbm.at[pl.ds(base, PPW)], b_v)
        acc_v[...] = jnp.zeros((CHUNK,), jnp.float32)

        def body(i, carry):
            sl = pl.ds(i * CHUNK, CHUNK)
            d = a_v[sl] - b_v[sl]
            acc_v[...] = acc_v[...] + d * d
            return carry

        lax.fori_loop(0, PPW // CHUNK, body, 0)
        pltpu.sync_copy(acc_v, part_hbm.at[wid])

    return sqdiff_kernel(a, b)


def _conv(x, w, b, stride, pad):
    y = lax.conv_general_dilated(x, w, (stride, stride), [(pad, pad), (pad, pad)],
                                 dimension_numbers=('NCHW', 'OIHW', 'NCHW'))
    return y + b[None, :, None, None]


def _conv_t(x, w, b, stride, pad):
    k = w.shape[2]
    w2 = jnp.flip(w, axis=(2, 3)).transpose(1, 0, 2, 3)
    p = k - 1 - pad
    y = lax.conv_general_dilated(x, w2, (1, 1), [(p, p), (p, p)],
                                 lhs_dilation=(stride, stride),
                                 dimension_numbers=('NCHW', 'OIHW', 'NCHW'))
    return y + b[None, :, None, None]


def _bn(x, g, b):
    m = jnp.mean(x, axis=(0, 2, 3), keepdims=True)
    v = jnp.var(x, axis=(0, 2, 3), keepdims=True)
    return (x - m) / jnp.sqrt(v + 1e-5) * g[None, :, None, None] + b[None, :, None, None]


def kernel(x, enc_w1, enc_b1, bn1_g, bn1_b, enc_w2, enc_b2, bn2_g, bn2_b,
           preq_w, preq_b, codebook, postq_w, postq_b, dec_w1, dec_b1,
           bn3_g, bn3_b, dec_w2, dec_b2):
    h = jax.nn.relu(_bn(_conv(x, enc_w1, enc_b1, 2, 1), bn1_g, bn1_b))
    h = jax.nn.relu(_bn(_conv(h, enc_w2, enc_b2, 2, 1), bn2_g, bn2_b))
    q_in = _conv(h, preq_w, preq_b, 1, 0)
    Bq, C, Hq, Wq = q_in.shape
    qf = q_in.transpose(0, 2, 3, 1).reshape(NQ, C)

    cbt = codebook.T
    idx = _vq_argmin(qf, cbt).reshape(NQ)

    pad = NPAD - NQ
    idx_p = jnp.concatenate([idx, jnp.zeros((pad,), jnp.int32)])
    qfx_p = jnp.concatenate([qf[:, 0], jnp.zeros((pad,), jnp.float32)])
    qfy_p = jnp.concatenate([qf[:, 1], jnp.zeros((pad,), jnp.float32)])
    qstx, qsty, cb_part = _sc_quantize(cbt[0], cbt[1], idx_p, qfx_p, qfy_p)
    qst = jnp.stack([qstx[:NQ], qsty[:NQ]], axis=-1)
    q_out = qst.reshape(Bq, Hq, Wq, C).transpose(0, 3, 1, 2)

    d = _conv(q_out, postq_w, postq_b, 1, 0)
    d = jax.nn.relu(_bn(_conv_t(d, dec_w1, dec_b1, 2, 1), bn3_g, bn3_b))
    out = jnp.tanh(_conv_t(d, dec_w2, dec_b2, 2, 1))

    recon = jnp.mean((out - x) ** 2)
    cb_loss = jnp.sum(cb_part) / jnp.float32(2 * NQ)
    total = recon + (cb_loss + BETA * cb_loss)
    return (out, total, (recon, cb_loss, cb_loss))


# speedup vs baseline: 1.5504x; 1.0008x over previous
import functools

import jax
import jax.numpy as jnp
from jax import lax
from jax.experimental import pallas as pl
from jax.experimental.pallas import tpu as pltpu
from jax.experimental.pallas import tpu_sc as plsc

BETA = 0.2
K = 8192
NQ = 12544
NT = 256
NW = 32
NPAD = 12800
BPW = NPAD // NW
CHUNK = 16


def _vq_argmin_body(qf_ref, cbt_ref, idx_ref):
    f32 = jnp.float32
    q = qf_ref[...]
    cbt = cbt_ref[...]
    cbx = cbt[0:1, :]
    cby = cbt[1:2, :]
    qx = q[:, 0:1]
    qy = q[:, 1:2]
    b2 = cbx * cbx + cby * cby
    a2 = qx * qx + qy * qy
    m = lax.dot_general(q.astype(jnp.bfloat16), cbt.astype(jnp.bfloat16),
                        (((1,), (0,)), ((), ())),
                        preferred_element_type=f32)
    dist = jnp.sqrt(jnp.maximum((a2 + b2) - 2.0 * m, 0.0))
    mn = jnp.min(dist, axis=1, keepdims=True)
    ii = lax.broadcasted_iota(jnp.int32, dist.shape, 1)
    idx = jnp.min(jnp.where(dist <= mn, ii, K), axis=1)
    idx_ref[...] = idx.astype(jnp.int32).reshape(NT, 1)


def _vq_argmin(qf, cbt):
    return pl.pallas_call(
        _vq_argmin_body,
        grid=(NQ // NT,),
        in_specs=[
            pl.BlockSpec((NT, 2), lambda i: (i, 0)),
            pl.BlockSpec((2, K), lambda i: (0, 0)),
        ],
        out_specs=pl.BlockSpec((NT, 1), lambda i: (i, 0)),
        out_shape=jax.ShapeDtypeStruct((NQ, 1), jnp.int32),
    )(qf, cbt)


def _sc_quantize(cbx, cby, idx, qfx, qfy):
    mesh = plsc.VectorSubcoreMesh(core_axis_name="c", subcore_axis_name="s")

    @functools.partial(
        pl.kernel,
        mesh=mesh,
        compiler_params=pltpu.CompilerParams(needs_layout_passes=False),
        out_type=[
            jax.ShapeDtypeStruct((NPAD,), jnp.float32),
            jax.ShapeDtypeStruct((NPAD,), jnp.float32),
            jax.ShapeDtypeStruct((NW, CHUNK), jnp.float32),
        ],
        scratch_types=[
            pltpu.VMEM((K,), jnp.float32),
            pltpu.VMEM((K,), jnp.float32),
            pltpu.VMEM((BPW,), jnp.int32),
            pltpu.VMEM((BPW,), jnp.float32),
            pltpu.VMEM((BPW,), jnp.float32),
            pltpu.VMEM((BPW,), jnp.float32),
            pltpu.VMEM((BPW,), jnp.float32),
            pltpu.VMEM((CHUNK,), jnp.float32),
        ],
    )
    def gather_kernel(cbx_hbm, cby_hbm, idx_hbm, qfx_hbm, qfy_hbm,
                      ox_hbm, oy_hbm, part_hbm,
                      cbx_v, cby_v, idx_v, qfx_v, qfy_v, ox_v, oy_v, acc_v):
        wid = lax.axis_index("s") * 2 + lax.axis_index("c")
        base = wid * BPW
        pltpu.sync_copy(cbx_hbm, cbx_v)
        pltpu.sync_copy(cby_hbm, cby_v)
        pltpu.sync_copy(idx_hbm.at[pl.ds(base, BPW)], idx_v)
        pltpu.sync_copy(qfx_hbm.at[pl.ds(base, BPW)], qfx_v)
        pltpu.sync_copy(qfy_hbm.at[pl.ds(base, BPW)], qfy_v)
        acc_v[...] = jnp.zeros((CHUNK,), jnp.float32)

        def body(i, carry):
            sl = pl.ds(i * CHUNK, CHUNK)
            iv = idx_v[sl]
            gx = plsc.load_gather(cbx_v, [iv])
            gy = plsc.load_gather(cby_v, [iv])
            fx = qfx_v[sl]
            fy = qfy_v[sl]
            valid = (base + i * CHUNK + lax.iota(jnp.int32, CHUNK)) < NQ
            zero = jnp.zeros((CHUNK,), jnp.float32)
            ex = jnp.where(valid, gx - fx, zero)
            ey = jnp.where(valid, gy - fy, zero)
            acc_v[...] = acc_v[...] + (ex * ex + ey * ey)
            ox_v[sl] = jnp.where(valid, fx + ex, zero)
            oy_v[sl] = jnp.where(valid, fy + ey, zero)
            return carry

        lax.fori_loop(0, BPW // CHUNK, body, 0)
        pltpu.sync_copy(ox_v, ox_hbm.at[pl.ds(base, BPW)])
        pltpu.sync_copy(oy_v, oy_hbm.at[pl.ds(base, BPW)])
        pltpu.sync_copy(acc_v, part_hbm.at[wid])

    return gather_kernel(cbx, cby, idx, qfx, qfy)


def _conv(x, w, b, stride, pad):
    y = lax.conv_general_dilated(x, w, (stride, stride), [(pad, pad), (pad, pad)],
                                 dimension_numbers=('NCHW', 'OIHW', 'NCHW'))
    return y + b[None, :, None, None]


def _conv_t(x, w, b, stride, pad):
    k = w.shape[2]
    w2 = jnp.flip(w, axis=(2, 3)).transpose(1, 0, 2, 3)
    p = k - 1 - pad
    y = lax.conv_general_dilated(x, w2, (1, 1), [(p, p), (p, p)],
                                 lhs_dilation=(stride, stride),
                                 dimension_numbers=('NCHW', 'OIHW', 'NCHW'))
    return y + b[None, :, None, None]


def _bn(x, g, b):
    m = jnp.mean(x, axis=(0, 2, 3), keepdims=True)
    v = jnp.var(x, axis=(0, 2, 3), keepdims=True)
    return (x - m) / jnp.sqrt(v + 1e-5) * g[None, :, None, None] + b[None, :, None, None]


def kernel(x, enc_w1, enc_b1, bn1_g, bn1_b, enc_w2, enc_b2, bn2_g, bn2_b,
           preq_w, preq_b, codebook, postq_w, postq_b, dec_w1, dec_b1,
           bn3_g, bn3_b, dec_w2, dec_b2):
    h = jax.nn.relu(_bn(_conv(x, enc_w1, enc_b1, 2, 1), bn1_g, bn1_b))
    h = jax.nn.relu(_bn(_conv(h, enc_w2, enc_b2, 2, 1), bn2_g, bn2_b))
    q_in = _conv(h, preq_w, preq_b, 1, 0)
    Bq, C, Hq, Wq = q_in.shape
    qf = q_in.transpose(0, 2, 3, 1).reshape(NQ, C)

    cbt = codebook.T
    idx = _vq_argmin(qf, cbt).reshape(NQ)

    pad = NPAD - NQ
    idx_p = jnp.concatenate([idx, jnp.zeros((pad,), jnp.int32)])
    qfx_p = jnp.concatenate([qf[:, 0], jnp.zeros((pad,), jnp.float32)])
    qfy_p = jnp.concatenate([qf[:, 1], jnp.zeros((pad,), jnp.float32)])
    qstx, qsty, cb_part = _sc_quantize(cbt[0], cbt[1], idx_p, qfx_p, qfy_p)
    qst = jnp.stack([qstx[:NQ], qsty[:NQ]], axis=-1)
    q_out = qst.reshape(Bq, Hq, Wq, C).transpose(0, 3, 1, 2)

    d = _conv(q_out, postq_w, postq_b, 1, 0)
    d = jax.nn.relu(_bn(_conv_t(d, dec_w1, dec_b1, 2, 1), bn3_g, bn3_b))
    out = jnp.tanh(_conv_t(d, dec_w2, dec_b2, 2, 1))

    recon = jnp.mean((out - x) ** 2)
    cb_loss = jnp.sum(cb_part) / jnp.float32(2 * NQ)
    total = recon + (cb_loss + BETA * cb_loss)
    return (out, total, (recon, cb_loss, cb_loss))
